# baseline (speedup 1.0000x reference)
import jax
import jax.numpy as jnp
from jax.experimental import pallas as pl
from jax.experimental.pallas import tpu as pltpu


_BR = 1024


def kernel(x, pos_table):
    B, S, D = x.shape
    R = B * S
    br = _BR if R % _BR == 0 and S % _BR == 0 else S
    xf = x.reshape(R, D)
    n_tab = S // br

    def body(x_ref, t_ref, o_ref):
        i = pl.program_id(0)
        off = (i % n_tab) * br
        o_ref[...] = x_ref[...] + t_ref[pl.ds(off, br), :]

    out = pl.pallas_call(
        body,
        grid=(R // br,),
        in_specs=[
            pl.BlockSpec((br, D), lambda i: (i, 0)),
            pl.BlockSpec((S, D), lambda i: (0, 0)),
        ],
        out_specs=pl.BlockSpec((br, D), lambda i: (i, 0)),
        out_shape=jax.ShapeDtypeStruct((R, D), x.dtype),
        compiler_params=pltpu.CompilerParams(
            dimension_semantics=("arbitrary",),
            vmem_limit_bytes=100 * 1024 * 1024,
        ),
    )(xf, pos_table)
    return out.reshape(B, S, D)
